# baseline (speedup 1.0000x reference)
import functools

import jax
import jax.numpy as jnp
from jax import lax
from jax.experimental import pallas as pl
from jax.experimental.pallas import tpu as pltpu
from jax.experimental.pallas import tpu_sc as plsc

TOKEN_SIZE = 23
D_MODEL = 1280
BATCH = 4
SEQ = 8192
N = BATCH * SEQ

NUM_CORES = 2
NUM_SUBCORES = 16
NW = NUM_CORES * NUM_SUBCORES
BPW = N // NW
R = 32
NCHUNK = BPW // R
REP_STRIDE = 24


def _build():
  mesh = plsc.VectorSubcoreMesh(core_axis_name="c", subcore_axis_name="s")

  @functools.partial(
      pl.kernel,
      mesh=mesh,
      out_type=(
          jax.ShapeDtypeStruct((N, D_MODEL), jnp.float32),
          jax.ShapeDtypeStruct((NW * REP_STRIDE, D_MODEL), jnp.float32),
      ),
      scratch_types=[
          pltpu.VMEM((NCHUNK, R), jnp.int32),
          pltpu.VMEM((REP_STRIDE, D_MODEL), jnp.float32),
          pltpu.VMEM((R, D_MODEL), jnp.float32),
          pltpu.VMEM((R, D_MODEL), jnp.float32),
          pltpu.SemaphoreType.DMA,
          pltpu.SemaphoreType.DMA,
          pltpu.SemaphoreType.DMA,
          pltpu.SemaphoreType.DMA,
      ],
  )
  def emb_kernel(idx_hbm, table_hbm, out_hbm, rep_hbm,
                 idx_v, tab_v, buf0, buf1, sg0, sg1, so0, so1):
    wid = lax.axis_index("s") * NUM_CORES + lax.axis_index("c")
    base = wid * BPW

    pltpu.sync_copy(idx_hbm.at[wid], idx_v)
    pltpu.sync_copy(table_hbm, tab_v)
    pltpu.sync_copy(tab_v, rep_hbm.at[pl.ds(wid * REP_STRIDE, REP_STRIDE)])

    bufs = (buf0, buf1)
    sg = (sg0, sg1)
    so = (so0, so1)

    def gather(c, j):
      pltpu.async_copy(rep_hbm.at[idx_v.at[c]], bufs[j], sg[j])

    def gather_wait(c, j):
      pltpu.make_async_copy(rep_hbm.at[idx_v.at[c]], bufs[j], sg[j]).wait()

    def put(c, j):
      pltpu.async_copy(bufs[j], out_hbm.at[pl.ds(base + c * R, R)], so[j])

    def put_wait(c, j):
      pltpu.make_async_copy(
          bufs[j], out_hbm.at[pl.ds(base + c * R, R)], so[j]).wait()

    gather(0, 0)

    def body(p, _):
      c0 = 2 * p
      gather_wait(c0, 0)
      put(c0, 0)

      @pl.when(c0 > 0)
      def _():
        put_wait(c0 - 1, 1)

      gather(c0 + 1, 1)
      gather_wait(c0 + 1, 1)
      put(c0 + 1, 1)
      put_wait(c0, 0)

      @pl.when(c0 + 2 < NCHUNK)
      def _():
        gather(c0 + 2, 0)

      return _

    lax.fori_loop(0, NCHUNK // 2, body, None)
    put_wait(NCHUNK - 1, 1)

  return emb_kernel


_emb = _build()


def kernel(x, embed_weight):
  idx = x.reshape(NW, NCHUNK, R).astype(jnp.int32)
  idx = idx + (jnp.arange(NW, dtype=jnp.int32) * REP_STRIDE)[:, None, None]
  table_pad = jnp.concatenate(
      [embed_weight,
       jnp.zeros((REP_STRIDE - TOKEN_SIZE, D_MODEL), jnp.float32)], axis=0)
  out, _ = _emb(idx, table_pad)
  return out.reshape(BATCH, SEQ, D_MODEL)
